# baseline (speedup 1.0000x reference)
import functools
import jax
import jax.numpy as jnp
from jax import lax
from jax.experimental import pallas as pl
from jax.experimental.pallas import tpu as pltpu
from jax.experimental.pallas import tpu_sc as plsc

N_PAD = 10240
NC = 2
NS = 16
CHUNK = 128
IDX_BLK = 16
ROWS_PER_TILE = N_PAD // NS
SPLIT_FRAC = 0.65


def _split(e):
    t = 8 * (-(-e // (NS * CHUNK * 8)))
    a = 8 * int(round(SPLIT_FRAC * t / 8))
    a = max(8, min(t - 8, a))
    while NS * a * CHUNK > e:
        a -= 8
    return a, t - a


def _grd(fn, c, core0_only):
    def run():
        fn()

    if core0_only:
        pl.when(c == 0)(run)
    else:
        fn()


def _deg_body(a, b, e0_hbm, e1_hbm, zeros_hbm, out_hbm, idx_v, ones_v, dacc,
              sem):
    c = lax.axis_index("c")
    s = lax.axis_index("s")
    for k in range(CHUNK // 16):
        ones_v[pl.ds(k * 16, 16)] = jnp.ones((16,), jnp.float32)
    pltpu.sync_copy(zeros_hbm.at[pl.ds(s * ROWS_PER_TILE, ROWS_PER_TILE)],
                    dacc.at[pl.ds(s * ROWS_PER_TILE, ROWS_PER_TILE)])

    def stage0():
        pltpu.sync_copy(e0_hbm.at[1, s], idx_v)

    def stage1():
        pltpu.sync_copy(e1_hbm.at[1, s], idx_v.at[pl.ds(0, b)])

    pl.when(c == 0)(stage0)
    pl.when(c == 1)(stage1)
    plsc.subcore_barrier()
    handles = {}
    for j in range(a):
        _grd(lambda j=j: handles.__setitem__(j, pltpu.async_copy(
            ones_v, dacc.at[idx_v.at[j]], sem, add=True)), c, j >= b)
    for j in range(a):
        _grd(lambda j=j: handles[j].wait(), c, j >= b)
    plsc.subcore_barrier()
    pltpu.sync_copy(dacc.at[pl.ds(s * ROWS_PER_TILE, ROWS_PER_TILE)],
                    out_hbm.at[c, pl.ds(s * ROWS_PER_TILE, ROWS_PER_TILE)])


def _agg_body(a, b, featn_hbm, e0_hbm, e1_hbm, zeros_hbm, out_hbm,
              sidx_v, didx_v, rows0, rows1, acc, gsem0, gsem1):
    c = lax.axis_index("c")
    s = lax.axis_index("s")
    pltpu.sync_copy(zeros_hbm.at[pl.ds(s * ROWS_PER_TILE, ROWS_PER_TILE)],
                    acc.at[pl.ds(s * ROWS_PER_TILE, ROWS_PER_TILE)])
    plsc.subcore_barrier()
    bufs = (rows0, rows1)
    gsems = (gsem0, gsem1)
    for blk in range(0, a, IDX_BLK):
        k0 = min(IDX_BLK, a - blk)

        def stage0(blk=blk, k0=k0):
            pltpu.sync_copy(e0_hbm.at[0, s, pl.ds(blk, k0)],
                            sidx_v.at[pl.ds(0, k0)])
            pltpu.sync_copy(e0_hbm.at[1, s, pl.ds(blk, k0)],
                            didx_v.at[pl.ds(0, k0)])

        pl.when(c == 0)(stage0)
        if blk < b:
            k1 = min(IDX_BLK, b - blk)

            def stage1(blk=blk, k1=k1):
                pltpu.sync_copy(e1_hbm.at[0, s, pl.ds(blk, k1)],
                                sidx_v.at[pl.ds(0, k1)])
                pltpu.sync_copy(e1_hbm.at[1, s, pl.ds(blk, k1)],
                                didx_v.at[pl.ds(0, k1)])

            pl.when(c == 1)(stage1)

        gh = {}

        def gather(j, g):
            gh[j] = pltpu.async_copy(
                featn_hbm.at[sidx_v.at[j]], bufs[g % 2], gsems[g % 2])

        _grd(lambda: gather(0, blk), c, blk >= b)
        for j in range(k0):
            g = blk + j
            if j + 1 < k0:
                _grd(lambda j=j, g=g: gather(j + 1, g + 1), c, g + 1 >= b)
            _grd(lambda j=j: gh[j].wait(), c, g >= b)
            _grd(lambda j=j, g=g: pltpu.sync_copy(
                bufs[g % 2], acc.at[didx_v.at[j]], add=True), c, g >= b)
    plsc.subcore_barrier()
    pltpu.sync_copy(acc.at[pl.ds(s * ROWS_PER_TILE, ROWS_PER_TILE)],
                    out_hbm.at[c, pl.ds(s * ROWS_PER_TILE, ROWS_PER_TILE)])


def _norm_scale_body(deg_ref, feat_ref, featn_ref, norm_ref):
    d = deg_ref[0] + deg_ref[1]
    norm = lax.rsqrt(jnp.maximum(d, 1.0))
    norm_ref[...] = norm
    featn_ref[...] = feat_ref[...] * norm


def _out_body(acc_ref, w_ref, norm_ref, bias_ref, out_ref):
    a = acc_ref[0] + acc_ref[1]
    y = jnp.dot(a, w_ref[...], preferred_element_type=jnp.float32)
    out_ref[...] = y * norm_ref[...] + bias_ref[...]


def kernel(feat, edge_index, weight, bias):
    n, d_in = feat.shape
    d_out = weight.shape[1]
    e = edge_index.shape[1]
    a, b = _split(e)
    nt = NS * (a + b)
    npad = nt * CHUNK - e

    if edge_index.dtype == jnp.int64:
        ei32 = lax.bitcast_convert_type(edge_index, jnp.int32)[..., 0]
    else:
        ei32 = edge_index.astype(jnp.int32)
    n0 = NS * a * CHUNK
    epad = jnp.concatenate([ei32, jnp.full((2, npad), n, jnp.int32)], axis=1)
    e0 = epad[:, :n0].reshape(2, NS, a, CHUNK)
    e1 = epad[:, n0:].reshape(2, NS, b, CHUNK)

    zeros2d = jnp.zeros((N_PAD, d_in), jnp.float32)
    zeros1d = jnp.zeros((N_PAD,), jnp.float32)

    mesh = plsc.VectorSubcoreMesh(core_axis_name="c", subcore_axis_name="s")

    deg2 = pl.kernel(
        functools.partial(_deg_body, a, b),
        out_type=jax.ShapeDtypeStruct((NC, N_PAD), jnp.float32),
        mesh=mesh,
        scratch_types=[
            pltpu.VMEM((a, CHUNK), jnp.int32),
            pltpu.VMEM((CHUNK,), jnp.float32),
            pltpu.VMEM_SHARED((N_PAD,), jnp.float32),
            pltpu.SemaphoreType.DMA,
        ],
    )(e0, e1, zeros1d)

    deg2 = deg2.reshape(NC, N_PAD, 1)

    blk = 1280
    grid = N_PAD // blk
    featn, norm = pl.pallas_call(
        _norm_scale_body,
        grid=(grid,),
        in_specs=[
            pl.BlockSpec((NC, blk, 1), lambda i: (0, i, 0)),
            pl.BlockSpec((blk, d_in), lambda i: (i, 0)),
        ],
        out_specs=[
            pl.BlockSpec((blk, d_in), lambda i: (i, 0)),
            pl.BlockSpec((blk, 1), lambda i: (i, 0)),
        ],
        out_shape=[
            jax.ShapeDtypeStruct((N_PAD, d_in), jnp.float32),
            jax.ShapeDtypeStruct((N_PAD, 1), jnp.float32),
        ],
    )(deg2, feat)

    acc2 = pl.kernel(
        functools.partial(_agg_body, a, b),
        out_type=jax.ShapeDtypeStruct((NC, N_PAD, d_in), jnp.float32),
        mesh=mesh,
        scratch_types=[
            pltpu.VMEM((IDX_BLK, CHUNK), jnp.int32),
            pltpu.VMEM((IDX_BLK, CHUNK), jnp.int32),
            pltpu.VMEM((CHUNK, d_in), jnp.float32),
            pltpu.VMEM((CHUNK, d_in), jnp.float32),
            pltpu.VMEM_SHARED((N_PAD, d_in), jnp.float32),
            pltpu.SemaphoreType.DMA,
            pltpu.SemaphoreType.DMA,
        ],
    )(featn, e0, e1, zeros2d)

    out = pl.pallas_call(
        _out_body,
        grid=(grid,),
        in_specs=[
            pl.BlockSpec((NC, blk, d_in), lambda i: (0, i, 0)),
            pl.BlockSpec((d_in, d_out), lambda i: (0, 0)),
            pl.BlockSpec((blk, 1), lambda i: (i, 0)),
            pl.BlockSpec((1, d_out), lambda i: (0, 0)),
        ],
        out_specs=pl.BlockSpec((blk, d_out), lambda i: (i, 0)),
        out_shape=jax.ShapeDtypeStruct((n, d_out), jnp.float32),
    )(acc2, weight, norm, bias.reshape(1, d_out))

    return out


# speedup vs baseline: 2.2428x; 2.2428x over previous
import functools
import jax
import jax.numpy as jnp
from jax import lax
from jax.experimental import pallas as pl
from jax.experimental.pallas import tpu as pltpu
from jax.experimental.pallas import tpu_sc as plsc

N_PAD = 10240
NC = 2
NS = 16
CHUNK = 128
IDX_BLK = 16
ROWS_PER_TILE = N_PAD // NS
SPLIT_FRAC = 0.68


def _split(e):
    t = -(-e // (NS * CHUNK))
    a = min(t, max(1, round(SPLIT_FRAC * t)))
    while NS * a * CHUNK > e:
        a -= 1
    return a, t - a


def _grd(fn, c, core0_only):
    def run():
        fn()

    if core0_only:
        pl.when(c == 0)(run)
    else:
        fn()


def _deg_body(a, b, e0_hbm, e1_hbm, zeros_hbm, out_hbm, idx_v, ones_v, dacc,
              sem):
    c = lax.axis_index("c")
    s = lax.axis_index("s")
    for k in range(CHUNK // 16):
        ones_v[pl.ds(k * 16, 16)] = jnp.ones((16,), jnp.float32)
    pltpu.sync_copy(zeros_hbm.at[pl.ds(s * ROWS_PER_TILE, ROWS_PER_TILE)],
                    dacc.at[pl.ds(s * ROWS_PER_TILE, ROWS_PER_TILE)])

    def stage0():
        pltpu.sync_copy(e0_hbm.at[1, s], idx_v)

    def stage1():
        pltpu.sync_copy(e1_hbm.at[1, s], idx_v.at[pl.ds(0, b)])

    pl.when(c == 0)(stage0)
    pl.when(c == 1)(stage1)
    plsc.subcore_barrier()
    for j in range(a):
        _grd(lambda j=j: pltpu.sync_copy(ones_v, dacc.at[idx_v.at[j]],
                                         add=True), c, j >= b)
    plsc.subcore_barrier()
    pltpu.sync_copy(dacc.at[pl.ds(s * ROWS_PER_TILE, ROWS_PER_TILE)],
                    out_hbm.at[c, pl.ds(s * ROWS_PER_TILE, ROWS_PER_TILE)])


def _agg_body(a, b, featn_hbm, e0_hbm, e1_hbm, zeros_hbm, out_hbm,
              sidx_v, didx_v, rows0, rows1, acc, gsem0, gsem1):
    c = lax.axis_index("c")
    s = lax.axis_index("s")
    pltpu.sync_copy(zeros_hbm.at[pl.ds(s * ROWS_PER_TILE, ROWS_PER_TILE)],
                    acc.at[pl.ds(s * ROWS_PER_TILE, ROWS_PER_TILE)])
    plsc.subcore_barrier()
    bufs = (rows0, rows1)
    gsems = (gsem0, gsem1)
    for blk in range(0, a, IDX_BLK):
        k0 = min(IDX_BLK, a - blk)

        def stage0(blk=blk, k0=k0):
            pltpu.sync_copy(e0_hbm.at[0, s, pl.ds(blk, k0)],
                            sidx_v.at[pl.ds(0, k0)])
            pltpu.sync_copy(e0_hbm.at[1, s, pl.ds(blk, k0)],
                            didx_v.at[pl.ds(0, k0)])

        pl.when(c == 0)(stage0)
        if blk < b:
            k1 = min(IDX_BLK, b - blk)

            def stage1(blk=blk, k1=k1):
                pltpu.sync_copy(e1_hbm.at[0, s, pl.ds(blk, k1)],
                                sidx_v.at[pl.ds(0, k1)])
                pltpu.sync_copy(e1_hbm.at[1, s, pl.ds(blk, k1)],
                                didx_v.at[pl.ds(0, k1)])

            pl.when(c == 1)(stage1)

        gh = {}

        def gather(j, g):
            gh[j] = pltpu.async_copy(
                featn_hbm.at[sidx_v.at[j]], bufs[g % 2], gsems[g % 2])

        _grd(lambda: gather(0, blk), c, blk >= b)
        for j in range(k0):
            g = blk + j
            if j + 1 < k0:
                _grd(lambda j=j, g=g: gather(j + 1, g + 1), c, g + 1 >= b)
            _grd(lambda j=j: gh[j].wait(), c, g >= b)
            _grd(lambda j=j, g=g: pltpu.sync_copy(
                bufs[g % 2], acc.at[didx_v.at[j]], add=True), c, g >= b)
    plsc.subcore_barrier()
    pltpu.sync_copy(acc.at[pl.ds(s * ROWS_PER_TILE, ROWS_PER_TILE)],
                    out_hbm.at[c, pl.ds(s * ROWS_PER_TILE, ROWS_PER_TILE)])


def _norm_scale_body(deg_ref, feat_ref, featn_ref, norm_ref):
    d = deg_ref[0] + deg_ref[1]
    norm = lax.rsqrt(jnp.maximum(d, 1.0))
    norm_ref[...] = norm
    featn_ref[...] = feat_ref[...] * norm


def _out_body(acc_ref, w_ref, norm_ref, bias_ref, out_ref):
    a = acc_ref[0] + acc_ref[1]
    y = jnp.dot(a, w_ref[...], preferred_element_type=jnp.float32)
    out_ref[...] = y * norm_ref[...] + bias_ref[...]


def kernel(feat, edge_index, weight, bias):
    n, d_in = feat.shape
    d_out = weight.shape[1]
    e = edge_index.shape[1]
    a, b = _split(e)
    nt = NS * (a + b)
    npad = nt * CHUNK - e

    if edge_index.dtype == jnp.int64:
        ei32 = lax.bitcast_convert_type(edge_index, jnp.int32)[..., 0]
    else:
        ei32 = edge_index.astype(jnp.int32)
    n0 = NS * a * CHUNK
    epad = jnp.concatenate([ei32, jnp.full((2, npad), n, jnp.int32)], axis=1)
    e0 = epad[:, :n0].reshape(2, NS, a, CHUNK)
    e1 = epad[:, n0:].reshape(2, NS, b, CHUNK)

    zeros2d = jnp.zeros((N_PAD, d_in), jnp.float32)
    zeros1d = jnp.zeros((N_PAD,), jnp.float32)

    mesh = plsc.VectorSubcoreMesh(core_axis_name="c", subcore_axis_name="s")

    deg2 = pl.kernel(
        functools.partial(_deg_body, a, b),
        out_type=jax.ShapeDtypeStruct((NC, N_PAD), jnp.float32),
        mesh=mesh,
        scratch_types=[
            pltpu.VMEM((a, CHUNK), jnp.int32),
            pltpu.VMEM((CHUNK,), jnp.float32),
            pltpu.VMEM_SHARED((N_PAD,), jnp.float32),
            pltpu.SemaphoreType.DMA,
        ],
    )(e0, e1, zeros1d)

    deg2 = deg2.reshape(NC, N_PAD, 1)

    blk = 1280
    grid = N_PAD // blk
    featn, norm = pl.pallas_call(
        _norm_scale_body,
        grid=(grid,),
        in_specs=[
            pl.BlockSpec((NC, blk, 1), lambda i: (0, i, 0)),
            pl.BlockSpec((blk, d_in), lambda i: (i, 0)),
        ],
        out_specs=[
            pl.BlockSpec((blk, d_in), lambda i: (i, 0)),
            pl.BlockSpec((blk, 1), lambda i: (i, 0)),
        ],
        out_shape=[
            jax.ShapeDtypeStruct((N_PAD, d_in), jnp.float32),
            jax.ShapeDtypeStruct((N_PAD, 1), jnp.float32),
        ],
    )(deg2, feat)

    acc2 = pl.kernel(
        functools.partial(_agg_body, a, b),
        out_type=jax.ShapeDtypeStruct((NC, N_PAD, d_in), jnp.float32),
        mesh=mesh,
        scratch_types=[
            pltpu.VMEM((IDX_BLK, CHUNK), jnp.int32),
            pltpu.VMEM((IDX_BLK, CHUNK), jnp.int32),
            pltpu.VMEM((CHUNK, d_in), jnp.float32),
            pltpu.VMEM((CHUNK, d_in), jnp.float32),
            pltpu.VMEM_SHARED((N_PAD, d_in), jnp.float32),
            pltpu.SemaphoreType.DMA,
            pltpu.SemaphoreType.DMA,
        ],
    )(featn, e0, e1, zeros2d)

    out = pl.pallas_call(
        _out_body,
        grid=(grid,),
        in_specs=[
            pl.BlockSpec((NC, blk, d_in), lambda i: (0, i, 0)),
            pl.BlockSpec((d_in, d_out), lambda i: (0, 0)),
            pl.BlockSpec((blk, 1), lambda i: (i, 0)),
            pl.BlockSpec((1, d_out), lambda i: (0, 0)),
        ],
        out_specs=pl.BlockSpec((blk, d_out), lambda i: (i, 0)),
        out_shape=jax.ShapeDtypeStruct((n, d_out), jnp.float32),
    )(acc2, weight, norm, bias.reshape(1, d_out))

    return out


# speedup vs baseline: 2.3570x; 1.0509x over previous
import functools
import jax
import jax.numpy as jnp
from jax import lax
from jax.experimental import pallas as pl
from jax.experimental.pallas import tpu as pltpu
from jax.experimental.pallas import tpu_sc as plsc

N_PAD = 10240
NC = 2
NS = 16
CHUNK = 128
IDX_BLK = 16
ROWS_PER_TILE = N_PAD // NS
SPLIT_FRAC = 0.666


def _split(e):
    t = -(-e // (NS * CHUNK))
    a = min(t, max(1, round(SPLIT_FRAC * t)))
    while NS * a * CHUNK > e:
        a -= 1
    return a, t - a


def _grd(fn, c, core0_only):
    def run():
        fn()

    if core0_only:
        pl.when(c == 0)(run)
    else:
        fn()


def _deg_body(a, b, e0_hbm, e1_hbm, zeros_hbm, out_hbm, idx_v, ones_v, dacc,
              sem):
    c = lax.axis_index("c")
    s = lax.axis_index("s")
    for k in range(CHUNK // 16):
        ones_v[pl.ds(k * 16, 16)] = jnp.ones((16,), jnp.float32)
    pltpu.sync_copy(zeros_hbm.at[pl.ds(s * ROWS_PER_TILE, ROWS_PER_TILE)],
                    dacc.at[pl.ds(s * ROWS_PER_TILE, ROWS_PER_TILE)])

    def stage0():
        pltpu.sync_copy(e0_hbm.at[1, s], idx_v)

    def stage1():
        pltpu.sync_copy(e1_hbm.at[1, s], idx_v.at[pl.ds(0, b)])

    pl.when(c == 0)(stage0)
    pl.when(c == 1)(stage1)
    plsc.subcore_barrier()
    for j in range(a):
        _grd(lambda j=j: pltpu.sync_copy(ones_v, dacc.at[idx_v.at[j]],
                                         add=True), c, j >= b)
    plsc.subcore_barrier()
    pltpu.sync_copy(dacc.at[pl.ds(s * ROWS_PER_TILE, ROWS_PER_TILE)],
                    out_hbm.at[c, pl.ds(s * ROWS_PER_TILE, ROWS_PER_TILE)])


def _agg_body(a, b, featn_hbm, e0_hbm, e1_hbm, zeros_hbm, out_hbm,
              sidx_v, didx_v, rows0, rows1, acc, gsem0, gsem1):
    c = lax.axis_index("c")
    s = lax.axis_index("s")
    pltpu.sync_copy(zeros_hbm.at[pl.ds(s * ROWS_PER_TILE, ROWS_PER_TILE)],
                    acc.at[pl.ds(s * ROWS_PER_TILE, ROWS_PER_TILE)])
    plsc.subcore_barrier()
    bufs = (rows0, rows1)
    gsems = (gsem0, gsem1)
    for blk in range(0, a, IDX_BLK):
        k0 = min(IDX_BLK, a - blk)

        def stage0(blk=blk, k0=k0):
            pltpu.sync_copy(e0_hbm.at[0, s, pl.ds(blk, k0)],
                            sidx_v.at[pl.ds(0, k0)])
            pltpu.sync_copy(e0_hbm.at[1, s, pl.ds(blk, k0)],
                            didx_v.at[pl.ds(0, k0)])

        pl.when(c == 0)(stage0)
        if blk < b:
            k1 = min(IDX_BLK, b - blk)

            def stage1(blk=blk, k1=k1):
                pltpu.sync_copy(e1_hbm.at[0, s, pl.ds(blk, k1)],
                                sidx_v.at[pl.ds(0, k1)])
                pltpu.sync_copy(e1_hbm.at[1, s, pl.ds(blk, k1)],
                                didx_v.at[pl.ds(0, k1)])

            pl.when(c == 1)(stage1)

        gh = {}

        def gather(j, g):
            gh[j] = pltpu.async_copy(
                featn_hbm.at[sidx_v.at[j]], bufs[g % 2], gsems[g % 2])

        _grd(lambda: gather(0, blk), c, blk >= b)
        for j in range(k0):
            g = blk + j
            if j + 1 < k0:
                _grd(lambda j=j, g=g: gather(j + 1, g + 1), c, g + 1 >= b)
            _grd(lambda j=j: gh[j].wait(), c, g >= b)
            _grd(lambda j=j, g=g: pltpu.sync_copy(
                bufs[g % 2], acc.at[didx_v.at[j]], add=True), c, g >= b)
    plsc.subcore_barrier()
    pltpu.sync_copy(acc.at[pl.ds(s * ROWS_PER_TILE, ROWS_PER_TILE)],
                    out_hbm.at[c, pl.ds(s * ROWS_PER_TILE, ROWS_PER_TILE)])


def _norm_scale_body(deg_ref, feat_ref, featn_ref, norm_ref):
    d = deg_ref[0] + deg_ref[1]
    norm = lax.rsqrt(jnp.maximum(d, 1.0))
    norm_ref[...] = norm
    featn_ref[...] = feat_ref[...] * norm


def _out_body(acc_ref, w_ref, norm_ref, bias_ref, out_ref):
    a = acc_ref[0] + acc_ref[1]
    y = jnp.dot(a, w_ref[...], preferred_element_type=jnp.float32)
    out_ref[...] = y * norm_ref[...] + bias_ref[...]


def kernel(feat, edge_index, weight, bias):
    n, d_in = feat.shape
    d_out = weight.shape[1]
    e = edge_index.shape[1]
    a, b = _split(e)
    nt = NS * (a + b)
    npad = nt * CHUNK - e

    if edge_index.dtype == jnp.int64:
        ei32 = lax.bitcast_convert_type(edge_index, jnp.int32)[..., 0]
    else:
        ei32 = edge_index.astype(jnp.int32)
    n0 = NS * a * CHUNK
    pad_src = (jnp.arange(npad, dtype=jnp.int32) * 37) % n
    pad_dst = n + (jnp.arange(npad, dtype=jnp.int32) % (N_PAD - n))
    epad = jnp.concatenate(
        [ei32, jnp.stack([pad_src, pad_dst])], axis=1)
    e0 = epad[:, :n0].reshape(2, NS, a, CHUNK)
    e1 = epad[:, n0:].reshape(2, NS, b, CHUNK)

    zeros2d = jnp.zeros((N_PAD, d_in), jnp.float32)
    zeros1d = jnp.zeros((N_PAD,), jnp.float32)

    mesh = plsc.VectorSubcoreMesh(core_axis_name="c", subcore_axis_name="s")

    deg2 = pl.kernel(
        functools.partial(_deg_body, a, b),
        out_type=jax.ShapeDtypeStruct((NC, N_PAD), jnp.float32),
        mesh=mesh,
        scratch_types=[
            pltpu.VMEM((a, CHUNK), jnp.int32),
            pltpu.VMEM((CHUNK,), jnp.float32),
            pltpu.VMEM_SHARED((N_PAD,), jnp.float32),
            pltpu.SemaphoreType.DMA,
        ],
    )(e0, e1, zeros1d)

    deg2 = deg2.reshape(NC, N_PAD, 1)

    blk = 1280
    grid = N_PAD // blk
    featn, norm = pl.pallas_call(
        _norm_scale_body,
        grid=(grid,),
        in_specs=[
            pl.BlockSpec((NC, blk, 1), lambda i: (0, i, 0)),
            pl.BlockSpec((blk, d_in), lambda i: (i, 0)),
        ],
        out_specs=[
            pl.BlockSpec((blk, d_in), lambda i: (i, 0)),
            pl.BlockSpec((blk, 1), lambda i: (i, 0)),
        ],
        out_shape=[
            jax.ShapeDtypeStruct((N_PAD, d_in), jnp.float32),
            jax.ShapeDtypeStruct((N_PAD, 1), jnp.float32),
        ],
    )(deg2, feat)

    acc2 = pl.kernel(
        functools.partial(_agg_body, a, b),
        out_type=jax.ShapeDtypeStruct((NC, N_PAD, d_in), jnp.float32),
        mesh=mesh,
        scratch_types=[
            pltpu.VMEM((IDX_BLK, CHUNK), jnp.int32),
            pltpu.VMEM((IDX_BLK, CHUNK), jnp.int32),
            pltpu.VMEM((CHUNK, d_in), jnp.float32),
            pltpu.VMEM((CHUNK, d_in), jnp.float32),
            pltpu.VMEM_SHARED((N_PAD, d_in), jnp.float32),
            pltpu.SemaphoreType.DMA,
            pltpu.SemaphoreType.DMA,
        ],
    )(featn, e0, e1, zeros2d)

    out = pl.pallas_call(
        _out_body,
        grid=(grid,),
        in_specs=[
            pl.BlockSpec((NC, blk, d_in), lambda i: (0, i, 0)),
            pl.BlockSpec((d_in, d_out), lambda i: (0, 0)),
            pl.BlockSpec((blk, 1), lambda i: (i, 0)),
            pl.BlockSpec((1, d_out), lambda i: (0, 0)),
        ],
        out_specs=pl.BlockSpec((blk, d_out), lambda i: (i, 0)),
        out_shape=jax.ShapeDtypeStruct((n, d_out), jnp.float32),
    )(acc2, weight, norm, bias.reshape(1, d_out))

    return out


# speedup vs baseline: 2.3871x; 1.0128x over previous
import functools
import jax
import jax.numpy as jnp
from jax import lax
from jax.experimental import pallas as pl
from jax.experimental.pallas import tpu as pltpu
from jax.experimental.pallas import tpu_sc as plsc

N_PAD = 10240
NC = 2
NS = 16
CHUNK = 128
IDX_BLK = 16
ROWS_PER_TILE = N_PAD // NS
SPLIT_FRAC = 0.666


def _split(e):
    t = -(-e // (NS * CHUNK))
    a = min(t, max(1, round(SPLIT_FRAC * t)))
    while NS * a * CHUNK > e:
        a -= 1
    return a, t - a


def _grd(fn, c, core0_only):
    def run():
        fn()

    if core0_only:
        pl.when(c == 0)(run)
    else:
        fn()


def _deg_body(a, b, e0_hbm, e1_hbm, zeros_hbm, out_hbm, idx_v, ones_v, dacc,
              sem):
    c = lax.axis_index("c")
    s = lax.axis_index("s")
    for k in range(CHUNK // 16):
        ones_v[pl.ds(k * 16, 16)] = jnp.ones((16,), jnp.float32)
    pltpu.sync_copy(zeros_hbm.at[pl.ds(s * ROWS_PER_TILE, ROWS_PER_TILE)],
                    dacc.at[pl.ds(s * ROWS_PER_TILE, ROWS_PER_TILE)])

    def stage0():
        pltpu.sync_copy(e0_hbm.at[1, s], idx_v)

    def stage1():
        pltpu.sync_copy(e1_hbm.at[1, s], idx_v.at[pl.ds(0, b)])

    pl.when(c == 0)(stage0)
    pl.when(c == 1)(stage1)
    plsc.subcore_barrier()
    handles = {}
    for j in range(a):
        _grd(lambda j=j: handles.__setitem__(j, pltpu.async_copy(
            ones_v, dacc.at[idx_v.at[j]], sem, add=True)), c, j >= b)
    for j in range(a):
        _grd(lambda j=j: handles[j].wait(), c, j >= b)
    plsc.subcore_barrier()
    pltpu.sync_copy(dacc.at[pl.ds(s * ROWS_PER_TILE, ROWS_PER_TILE)],
                    out_hbm.at[c, pl.ds(s * ROWS_PER_TILE, ROWS_PER_TILE)])


def _agg_body(a, b, featn_hbm, e0_hbm, e1_hbm, zeros_hbm, out_hbm,
              sidx_v, didx_v, rows0, rows1, acc, gsem0, gsem1):
    c = lax.axis_index("c")
    s = lax.axis_index("s")
    pltpu.sync_copy(zeros_hbm.at[pl.ds(s * ROWS_PER_TILE, ROWS_PER_TILE)],
                    acc.at[pl.ds(s * ROWS_PER_TILE, ROWS_PER_TILE)])
    plsc.subcore_barrier()
    bufs = (rows0, rows1)
    gsems = (gsem0, gsem1)
    for blk in range(0, a, IDX_BLK):
        k0 = min(IDX_BLK, a - blk)

        def stage0(blk=blk, k0=k0):
            pltpu.sync_copy(e0_hbm.at[0, s, pl.ds(blk, k0)],
                            sidx_v.at[pl.ds(0, k0)])
            pltpu.sync_copy(e0_hbm.at[1, s, pl.ds(blk, k0)],
                            didx_v.at[pl.ds(0, k0)])

        pl.when(c == 0)(stage0)
        if blk < b:
            k1 = min(IDX_BLK, b - blk)

            def stage1(blk=blk, k1=k1):
                pltpu.sync_copy(e1_hbm.at[0, s, pl.ds(blk, k1)],
                                sidx_v.at[pl.ds(0, k1)])
                pltpu.sync_copy(e1_hbm.at[1, s, pl.ds(blk, k1)],
                                didx_v.at[pl.ds(0, k1)])

            pl.when(c == 1)(stage1)

        gh = {}

        def gather(j, g):
            gh[j] = pltpu.async_copy(
                featn_hbm.at[sidx_v.at[j]], bufs[g % 2], gsems[g % 2])

        _grd(lambda: gather(0, blk), c, blk >= b)
        for j in range(k0):
            g = blk + j
            if j + 1 < k0:
                _grd(lambda j=j, g=g: gather(j + 1, g + 1), c, g + 1 >= b)
            _grd(lambda j=j: gh[j].wait(), c, g >= b)
            _grd(lambda j=j, g=g: pltpu.sync_copy(
                bufs[g % 2], acc.at[didx_v.at[j]], add=True), c, g >= b)
    plsc.subcore_barrier()
    pltpu.sync_copy(acc.at[pl.ds(s * ROWS_PER_TILE, ROWS_PER_TILE)],
                    out_hbm.at[c, pl.ds(s * ROWS_PER_TILE, ROWS_PER_TILE)])


def _norm_scale_body(deg_ref, feat_ref, featn_ref, norm_ref):
    d = deg_ref[0] + deg_ref[1]
    norm = lax.rsqrt(jnp.maximum(d, 1.0))
    norm_ref[...] = norm
    featn_ref[...] = feat_ref[...] * norm


def _out_body(acc_ref, w_ref, norm_ref, bias_ref, out_ref):
    a = acc_ref[0] + acc_ref[1]
    y = jnp.dot(a, w_ref[...], preferred_element_type=jnp.float32)
    out_ref[...] = y * norm_ref[...] + bias_ref[...]


def kernel(feat, edge_index, weight, bias):
    n, d_in = feat.shape
    d_out = weight.shape[1]
    e = edge_index.shape[1]
    a, b = _split(e)
    nt = NS * (a + b)
    npad = nt * CHUNK - e

    if edge_index.dtype == jnp.int64:
        ei32 = lax.bitcast_convert_type(edge_index, jnp.int32)[..., 0]
    else:
        ei32 = edge_index.astype(jnp.int32)
    n0 = NS * a * CHUNK
    pad_src = (jnp.arange(npad, dtype=jnp.int32) * 37) % n
    pad_dst = n + (jnp.arange(npad, dtype=jnp.int32) % (N_PAD - n))
    epad = jnp.concatenate(
        [ei32, jnp.stack([pad_src, pad_dst])], axis=1)
    e0 = epad[:, :n0].reshape(2, NS, a, CHUNK)
    e1 = epad[:, n0:].reshape(2, NS, b, CHUNK)

    zeros2d = jnp.zeros((N_PAD, d_in), jnp.float32)
    zeros1d = jnp.zeros((N_PAD,), jnp.float32)

    mesh = plsc.VectorSubcoreMesh(core_axis_name="c", subcore_axis_name="s")

    deg2 = pl.kernel(
        functools.partial(_deg_body, a, b),
        out_type=jax.ShapeDtypeStruct((NC, N_PAD), jnp.float32),
        mesh=mesh,
        scratch_types=[
            pltpu.VMEM((a, CHUNK), jnp.int32),
            pltpu.VMEM((CHUNK,), jnp.float32),
            pltpu.VMEM_SHARED((N_PAD,), jnp.float32),
            pltpu.SemaphoreType.DMA,
        ],
    )(e0, e1, zeros1d)

    deg2 = deg2.reshape(NC, N_PAD, 1)

    blk = 1280
    grid = N_PAD // blk
    featn, norm = pl.pallas_call(
        _norm_scale_body,
        grid=(grid,),
        in_specs=[
            pl.BlockSpec((NC, blk, 1), lambda i: (0, i, 0)),
            pl.BlockSpec((blk, d_in), lambda i: (i, 0)),
        ],
        out_specs=[
            pl.BlockSpec((blk, d_in), lambda i: (i, 0)),
            pl.BlockSpec((blk, 1), lambda i: (i, 0)),
        ],
        out_shape=[
            jax.ShapeDtypeStruct((N_PAD, d_in), jnp.float32),
            jax.ShapeDtypeStruct((N_PAD, 1), jnp.float32),
        ],
    )(deg2, feat)

    acc2 = pl.kernel(
        functools.partial(_agg_body, a, b),
        out_type=jax.ShapeDtypeStruct((NC, N_PAD, d_in), jnp.float32),
        mesh=mesh,
        scratch_types=[
            pltpu.VMEM((IDX_BLK, CHUNK), jnp.int32),
            pltpu.VMEM((IDX_BLK, CHUNK), jnp.int32),
            pltpu.VMEM((CHUNK, d_in), jnp.float32),
            pltpu.VMEM((CHUNK, d_in), jnp.float32),
            pltpu.VMEM_SHARED((N_PAD, d_in), jnp.float32),
            pltpu.SemaphoreType.DMA,
            pltpu.SemaphoreType.DMA,
        ],
    )(featn, e0, e1, zeros2d)

    out = pl.pallas_call(
        _out_body,
        grid=(grid,),
        in_specs=[
            pl.BlockSpec((NC, blk, d_in), lambda i: (0, i, 0)),
            pl.BlockSpec((d_in, d_out), lambda i: (0, 0)),
            pl.BlockSpec((blk, 1), lambda i: (i, 0)),
            pl.BlockSpec((1, d_out), lambda i: (0, 0)),
        ],
        out_specs=pl.BlockSpec((blk, d_out), lambda i: (i, 0)),
        out_shape=jax.ShapeDtypeStruct((n, d_out), jnp.float32),
    )(acc2, weight, norm, bias.reshape(1, d_out))

    return out


# speedup vs baseline: 2.4594x; 1.0303x over previous
import functools
import jax
import jax.numpy as jnp
from jax import lax
from jax.experimental import pallas as pl
from jax.experimental.pallas import tpu as pltpu
from jax.experimental.pallas import tpu_sc as plsc

N_PAD = 10240
NC = 2
NS = 16
CHUNK = 128
IDX_BLK = 16
ROWS_PER_TILE = N_PAD // NS
SPLIT_FRAC = 0.666


def _split(e):
    t = -(-e // (NS * CHUNK))
    a = min(t, max(1, round(SPLIT_FRAC * t)))
    while NS * a * CHUNK > e:
        a -= 1
    return a, t - a


def _grd(fn, c, core0_only):
    def run():
        fn()

    if core0_only:
        pl.when(c == 0)(run)
    else:
        fn()


def _deg_body(a, b, e0_hbm, e1_hbm, out_hbm, idx_v, ones_v, zvec, dacc, sem):
    c = lax.axis_index("c")
    s = lax.axis_index("s")
    for k in range(CHUNK // 16):
        ones_v[pl.ds(k * 16, 16)] = jnp.ones((16,), jnp.float32)
    for k in range(ROWS_PER_TILE // 16):
        zvec[pl.ds(k * 16, 16)] = jnp.zeros((16,), jnp.float32)
    pltpu.sync_copy(zvec,
                    dacc.at[pl.ds(s * ROWS_PER_TILE, ROWS_PER_TILE)])

    def stage0():
        pltpu.sync_copy(e0_hbm.at[1, s], idx_v)

    def stage1():
        pltpu.sync_copy(e1_hbm.at[1, s], idx_v.at[pl.ds(0, b)])

    pl.when(c == 0)(stage0)
    pl.when(c == 1)(stage1)
    plsc.subcore_barrier()
    handles = {}
    for j in range(a):
        _grd(lambda j=j: handles.__setitem__(j, pltpu.async_copy(
            ones_v, dacc.at[idx_v.at[j]], sem, add=True)), c, j >= b)
    for j in range(a):
        _grd(lambda j=j: handles[j].wait(), c, j >= b)
    plsc.subcore_barrier()
    pltpu.sync_copy(dacc.at[pl.ds(s * ROWS_PER_TILE, ROWS_PER_TILE)],
                    out_hbm.at[c, pl.ds(s * ROWS_PER_TILE, ROWS_PER_TILE)])


def _agg_body(a, b, featn_hbm, e0_hbm, e1_hbm, out_hbm,
              sidx_v, didx_v, rows0, rows1, acc, gsem0, gsem1):
    c = lax.axis_index("c")
    s = lax.axis_index("s")
    for k in range(CHUNK):
        for q in range(CHUNK // 16):
            rows0[k, pl.ds(q * 16, 16)] = jnp.zeros((16,), jnp.float32)
    for r in range(ROWS_PER_TILE // CHUNK):
        pltpu.sync_copy(
            rows0,
            acc.at[pl.ds(s * ROWS_PER_TILE + r * CHUNK, CHUNK)])
    plsc.subcore_barrier()
    bufs = (rows0, rows1)
    gsems = (gsem0, gsem1)
    for blk in range(0, a, IDX_BLK):
        k0 = min(IDX_BLK, a - blk)

        def stage0(blk=blk, k0=k0):
            pltpu.sync_copy(e0_hbm.at[0, s, pl.ds(blk, k0)],
                            sidx_v.at[pl.ds(0, k0)])
            pltpu.sync_copy(e0_hbm.at[1, s, pl.ds(blk, k0)],
                            didx_v.at[pl.ds(0, k0)])

        pl.when(c == 0)(stage0)
        if blk < b:
            k1 = min(IDX_BLK, b - blk)

            def stage1(blk=blk, k1=k1):
                pltpu.sync_copy(e1_hbm.at[0, s, pl.ds(blk, k1)],
                                sidx_v.at[pl.ds(0, k1)])
                pltpu.sync_copy(e1_hbm.at[1, s, pl.ds(blk, k1)],
                                didx_v.at[pl.ds(0, k1)])

            pl.when(c == 1)(stage1)

        gh = {}

        def gather(j, g):
            gh[j] = pltpu.async_copy(
                featn_hbm.at[sidx_v.at[j]], bufs[g % 2], gsems[g % 2])

        _grd(lambda: gather(0, blk), c, blk >= b)
        for j in range(k0):
            g = blk + j
            if j + 1 < k0:
                _grd(lambda j=j, g=g: gather(j + 1, g + 1), c, g + 1 >= b)
            _grd(lambda j=j: gh[j].wait(), c, g >= b)
            _grd(lambda j=j, g=g: pltpu.sync_copy(
                bufs[g % 2], acc.at[didx_v.at[j]], add=True), c, g >= b)
    plsc.subcore_barrier()
    pltpu.sync_copy(acc.at[pl.ds(s * ROWS_PER_TILE, ROWS_PER_TILE)],
                    out_hbm.at[c, pl.ds(s * ROWS_PER_TILE, ROWS_PER_TILE)])


def _norm_scale_body(deg_ref, feat_ref, featn_ref, norm_ref):
    d = deg_ref[0] + deg_ref[1]
    norm = lax.rsqrt(jnp.maximum(d, 1.0))
    norm_ref[...] = norm
    featn_ref[...] = feat_ref[...] * norm


def _out_body(acc_ref, w_ref, norm_ref, bias_ref, out_ref):
    a = acc_ref[0] + acc_ref[1]
    y = jnp.dot(a, w_ref[...], preferred_element_type=jnp.float32)
    out_ref[...] = y * norm_ref[...] + bias_ref[...]


def kernel(feat, edge_index, weight, bias):
    n, d_in = feat.shape
    d_out = weight.shape[1]
    e = edge_index.shape[1]
    a, b = _split(e)
    nt = NS * (a + b)
    npad = nt * CHUNK - e

    if edge_index.dtype == jnp.int64:
        ei32 = lax.bitcast_convert_type(edge_index, jnp.int32)[..., 0]
    else:
        ei32 = edge_index.astype(jnp.int32)
    n0 = NS * a * CHUNK
    pad_src = (jnp.arange(npad, dtype=jnp.int32) * 37) % n
    pad_dst = n + (jnp.arange(npad, dtype=jnp.int32) % (N_PAD - n))
    epad = jnp.concatenate(
        [ei32, jnp.stack([pad_src, pad_dst])], axis=1)
    e0 = epad[:, :n0].reshape(2, NS, a, CHUNK)
    e1 = epad[:, n0:].reshape(2, NS, b, CHUNK)

    mesh = plsc.VectorSubcoreMesh(core_axis_name="c", subcore_axis_name="s")

    deg2 = pl.kernel(
        functools.partial(_deg_body, a, b),
        out_type=jax.ShapeDtypeStruct((NC, N_PAD), jnp.float32),
        mesh=mesh,
        scratch_types=[
            pltpu.VMEM((a, CHUNK), jnp.int32),
            pltpu.VMEM((CHUNK,), jnp.float32),
            pltpu.VMEM((ROWS_PER_TILE,), jnp.float32),
            pltpu.VMEM_SHARED((N_PAD,), jnp.float32),
            pltpu.SemaphoreType.DMA,
        ],
    )(e0, e1)

    deg2 = deg2.reshape(NC, N_PAD, 1)

    blk = 1280
    grid = N_PAD // blk
    featn, norm = pl.pallas_call(
        _norm_scale_body,
        grid=(grid,),
        in_specs=[
            pl.BlockSpec((NC, blk, 1), lambda i: (0, i, 0)),
            pl.BlockSpec((blk, d_in), lambda i: (i, 0)),
        ],
        out_specs=[
            pl.BlockSpec((blk, d_in), lambda i: (i, 0)),
            pl.BlockSpec((blk, 1), lambda i: (i, 0)),
        ],
        out_shape=[
            jax.ShapeDtypeStruct((N_PAD, d_in), jnp.float32),
            jax.ShapeDtypeStruct((N_PAD, 1), jnp.float32),
        ],
    )(deg2, feat)

    acc2 = pl.kernel(
        functools.partial(_agg_body, a, b),
        out_type=jax.ShapeDtypeStruct((NC, N_PAD, d_in), jnp.float32),
        mesh=mesh,
        scratch_types=[
            pltpu.VMEM((IDX_BLK, CHUNK), jnp.int32),
            pltpu.VMEM((IDX_BLK, CHUNK), jnp.int32),
            pltpu.VMEM((CHUNK, d_in), jnp.float32),
            pltpu.VMEM((CHUNK, d_in), jnp.float32),
            pltpu.VMEM_SHARED((N_PAD, d_in), jnp.float32),
            pltpu.SemaphoreType.DMA,
            pltpu.SemaphoreType.DMA,
        ],
    )(featn, e0, e1)

    out = pl.pallas_call(
        _out_body,
        grid=(grid,),
        in_specs=[
            pl.BlockSpec((NC, blk, d_in), lambda i: (0, i, 0)),
            pl.BlockSpec((d_in, d_out), lambda i: (0, 0)),
            pl.BlockSpec((blk, 1), lambda i: (i, 0)),
            pl.BlockSpec((1, d_out), lambda i: (0, 0)),
        ],
        out_specs=pl.BlockSpec((blk, d_out), lambda i: (i, 0)),
        out_shape=jax.ShapeDtypeStruct((n, d_out), jnp.float32),
    )(acc2, weight, norm, bias.reshape(1, d_out))

    return out
